# baseline (device time: 359749 ns/iter reference)
import functools

import jax
import jax.numpy as jnp
from jax import lax
from jax.experimental import pallas as pl
from jax.experimental.pallas import tpu as pltpu

N_DEV = 8
HQ = 8
DH = 128
SQ = 256
SKV = 4096
QBLK = 64
NBLK = SKV // QBLK
CLS = 22
SLAB = CLS * QBLK
PAD0 = (CLS - 1) * QBLK
NKP = 3 * SLAB
EXT = 5 * QBLK
SCALE = 0.08838834764831843
NEG = -1e9

RING = (0, 1, 2, 3, 7, 6, 5, 4)
NEXT = (1, 2, 3, 7, 0, 4, 5, 6)
PREV = (4, 0, 1, 2, 5, 6, 7, 3)
POS = (0, 1, 2, 3, 7, 6, 5, 4)


def _body(meta_ref, q_ref, kp_ref, vp_ref, kr_ref, vr_ref, out_ref,
          qs_ref, acc_ref, st_ref, pv_ref, stl_ref, se_ref, me_ref,
          kpbuf, vpbuf, ekbuf, evbuf,
          qsend, qrecv, asend, arecv, ssend, srecv,
          kpsem, vpsem, eksem, evsem):
    right = meta_ref[8]
    left = meta_ref[9]
    my = meta_ref[10]
    my_mod3 = lax.rem(my, 3)

    qs_ref[0] = q_ref[...]
    acc_ref[0] = jnp.zeros((HQ, SQ, DH), jnp.bfloat16)
    st_ref[0, 0] = jnp.full((SQ, HQ), -1e30, jnp.float32)
    st_ref[0, 1] = jnp.zeros((SQ, HQ), jnp.float32)

    barrier = pltpu.get_barrier_semaphore()
    for nbr in (left, right):
        pl.semaphore_signal(barrier, inc=1, device_id=(nbr,),
                            device_id_type=pl.DeviceIdType.MESH)
    pl.semaphore_wait(barrier, 2)

    row_blk = lax.broadcasted_iota(jnp.int32, (SQ, 1), 0) // QBLK
    col_e = lax.broadcasted_iota(jnp.int32, (1, EXT), 1)
    colblk_e = col_e // QBLK
    pad_col = lax.broadcasted_iota(jnp.int32, (1, SLAB), 1)

    def _mk(src, dst, ssem, rsem, dev):
        return pltpu.make_async_remote_copy(
            src_ref=src, dst_ref=dst, send_sem=ssem, recv_sem=rsem,
            device_id=(dev,), device_id_type=pl.DeviceIdType.MESH)

    def _q_hop(s, nxt):
        return _mk(qs_ref.at[s], qs_ref.at[nxt], qsend.at[s],
                   qrecv.at[nxt], right)

    def _acc_hop(s, nxt):
        return (_mk(acc_ref.at[s], acc_ref.at[nxt], asend.at[s],
                    arecv.at[nxt], right),
                _mk(st_ref.at[s], st_ref.at[nxt], ssend.at[s],
                    srecv.at[nxt], right))

    def step(s, carry):
        nxt = (s + 1) % N_DEV
        c = meta_ref[s]

        @pl.when(s > 0)
        def _():
            _q_hop(s - 1, s).wait_send()
            _q_hop(s, s).wait_recv()

        @pl.when(s < N_DEV - 1)
        def _():
            _q_hop(s, nxt).start()

        def _fetch(h, b):
            return (
                pltpu.make_async_copy(kp_ref.at[h], kpbuf.at[b],
                                      kpsem.at[b]),
                pltpu.make_async_copy(vp_ref.at[h], vpbuf.at[b],
                                      vpsem.at[b]),
                pltpu.make_async_copy(kr_ref.at[h, pl.ds(SQ * c, SQ)],
                                      ekbuf.at[b, pl.ds(0, SQ)],
                                      eksem.at[b]),
                pltpu.make_async_copy(kr_ref.at[h, pl.ds(0, QBLK)],
                                      ekbuf.at[b, pl.ds(SQ, QBLK)],
                                      eksem.at[b]),
                pltpu.make_async_copy(vr_ref.at[h, pl.ds(SQ * c, SQ)],
                                      evbuf.at[b, pl.ds(0, SQ)],
                                      evsem.at[b]),
                pltpu.make_async_copy(vr_ref.at[h, pl.ds(0, QBLK)],
                                      evbuf.at[b, pl.ds(SQ, QBLK)],
                                      evsem.at[b]),
            )

        for r in _fetch(0, 0):
            r.start()

        on_dev0 = my == 0
        is_diag = colblk_e < 4
        keep_d = is_diag & (row_blk == colblk_e) & ((c + colblk_e) % 3 != 0)
        keep_0 = (~is_diag) & ((c + row_blk) % 3 != 0)
        bias_e = jnp.where(on_dev0 & (keep_d | keep_0), 0.0, NEG)

        for h in range(HQ):
            b = h % 2
            for r in _fetch(h, b):
                r.wait()
            if h + 1 < HQ:
                for r in _fetch(h + 1, (h + 1) % 2):
                    r.start()

            s_e = lax.dot_general(
                qs_ref[s, h], ekbuf[b], (((1,), (1,)), ((), ())),
                preferred_element_type=jnp.float32)
            s_e = s_e * SCALE + bias_e
            se_ref[...] = s_e
            me_ref[...] = jnp.max(s_e, axis=1, keepdims=True)

            def jbody(j, _, h=h, b=b):
                rj = (3 - (c + j) % 3) % 3
                off = rj * SLAB
                qj = qs_ref[s, h, pl.ds(QBLK * j, QBLK)]
                sm = lax.dot_general(
                    qj, kpbuf[b, pl.ds(off, SLAB)],
                    (((1,), (1,)), ((), ())),
                    preferred_element_type=jnp.float32)
                sm = sm * SCALE + jnp.where(
                    (pad_col < PAD0) | (rj == my_mod3), 0.0, NEG)
                me_j = me_ref[pl.ds(QBLK * j, QBLK)]
                se_j = se_ref[pl.ds(QBLK * j, QBLK)]
                m_j = jnp.maximum(jnp.max(sm, axis=1, keepdims=True), me_j)
                pm = jnp.exp(sm - m_j)
                pe = jnp.exp(se_j - m_j)
                l_j = (jnp.sum(pm, axis=1, keepdims=True)
                       + jnp.sum(pe, axis=1, keepdims=True))
                pv = lax.dot_general(
                    pm.astype(jnp.bfloat16), vpbuf[b, pl.ds(off, SLAB)],
                    (((1,), (0,)), ((), ())),
                    preferred_element_type=jnp.float32)
                pv = pv + lax.dot_general(
                    pe.astype(jnp.bfloat16), evbuf[b],
                    (((1,), (0,)), ((), ())),
                    preferred_element_type=jnp.float32)
                pv_ref[h, pl.ds(QBLK * j, QBLK)] = pv.astype(jnp.bfloat16)
                stl_ref[0, pl.ds(QBLK * j, QBLK), h:h + 1] = m_j
                stl_ref[1, pl.ds(QBLK * j, QBLK), h:h + 1] = l_j
                return _

            lax.fori_loop(0, 4, jbody, 0)

        @pl.when(s > 0)
        def _():
            for r in _acc_hop(s - 1, s):
                r.wait_send()
            for r in _acc_hop(s, s):
                r.wait_recv()

        for h in range(HQ):
            m_in = st_ref[s, 0, :, h:h + 1]
            l_in = st_ref[s, 1, :, h:h + 1]
            m_loc = stl_ref[0, :, h:h + 1]
            l_loc = stl_ref[1, :, h:h + 1]
            m_new = jnp.maximum(m_in, m_loc)
            a_in = jnp.exp(m_in - m_new)
            a_loc = jnp.exp(m_loc - m_new)
            st_ref[s, 0, :, h:h + 1] = m_new
            st_ref[s, 1, :, h:h + 1] = l_in * a_in + l_loc * a_loc
            acc_ref[s, h] = (
                acc_ref[s, h].astype(jnp.float32) * a_in
                + pv_ref[h].astype(jnp.float32) * a_loc
            ).astype(jnp.bfloat16)

        for r in _acc_hop(s, nxt):
            r.start()
        return carry

    lax.fori_loop(0, N_DEV, step, 0)

    for r in _acc_hop(N_DEV - 1, 0):
        r.wait_send()
    for r in _acc_hop(0, 0):
        r.wait_recv()

    for h in range(HQ):
        out_ref[h] = (acc_ref[0, h].astype(jnp.float32)
                      / st_ref[0, 1, :, h:h + 1])

    @functools.partial(pl.run_scoped, exit_sem=pltpu.SemaphoreType.REGULAR)
    def _(exit_sem):
        for nbr in (left, right):
            pl.semaphore_signal(exit_sem, inc=1, device_id=(nbr,),
                                device_id_type=pl.DeviceIdType.MESH)
        pl.semaphore_wait(exit_sem, 2)


def kernel(x, Wq, K_ext, V_ext, Wo):
    q = (x[0] @ Wq).reshape(SQ, HQ, DH).transpose(1, 0, 2)
    q = q.astype(jnp.bfloat16)
    k = K_ext[0].transpose(1, 0, 2).astype(jnp.bfloat16)
    v = V_ext[0].transpose(1, 0, 2).astype(jnp.bfloat16)

    my = lax.axis_index("i")
    ring = jnp.asarray(RING, jnp.int32)
    pos = jnp.asarray(POS, jnp.int32)[my]
    owners = ring[(pos - jnp.arange(N_DEV, dtype=jnp.int32)) % N_DEV]
    meta = jnp.concatenate([
        owners,
        jnp.asarray(NEXT, jnp.int32)[my][None],
        jnp.asarray(PREV, jnp.int32)[my][None],
        my.astype(jnp.int32)[None],
    ])

    t = jnp.arange(3 * CLS, dtype=jnp.int32)
    ct, it = t // CLS, t % CLS
    blk = jnp.minimum((ct - my) % 3 + 3 * it, NBLK - 1)
    rows = (blk[:, None] * QBLK
            + jnp.arange(QBLK, dtype=jnp.int32)[None, :]).reshape(-1)
    kp = jnp.take(k, rows, axis=1)
    vp = jnp.take(v, rows, axis=1)

    ctx = pl.pallas_call(
        _body,
        out_shape=jax.ShapeDtypeStruct((HQ, SQ, DH), jnp.float32),
        in_specs=[
            pl.BlockSpec(memory_space=pltpu.SMEM),
            pl.BlockSpec(memory_space=pltpu.VMEM),
            pl.BlockSpec(memory_space=pltpu.MemorySpace.HBM),
            pl.BlockSpec(memory_space=pltpu.MemorySpace.HBM),
            pl.BlockSpec(memory_space=pltpu.MemorySpace.HBM),
            pl.BlockSpec(memory_space=pltpu.MemorySpace.HBM),
        ],
        out_specs=pl.BlockSpec(memory_space=pltpu.VMEM),
        scratch_shapes=[
            pltpu.VMEM((N_DEV, HQ, SQ, DH), jnp.bfloat16),
            pltpu.VMEM((N_DEV, HQ, SQ, DH), jnp.bfloat16),
            pltpu.VMEM((N_DEV, 2, SQ, HQ), jnp.float32),
            pltpu.VMEM((HQ, SQ, DH), jnp.bfloat16),
            pltpu.VMEM((2, SQ, HQ), jnp.float32),
            pltpu.VMEM((SQ, EXT), jnp.float32),
            pltpu.VMEM((SQ, 1), jnp.float32),
            pltpu.VMEM((2, NKP, DH), jnp.bfloat16),
            pltpu.VMEM((2, NKP, DH), jnp.bfloat16),
            pltpu.VMEM((2, EXT, DH), jnp.bfloat16),
            pltpu.VMEM((2, EXT, DH), jnp.bfloat16),
            pltpu.SemaphoreType.DMA((N_DEV,)),
            pltpu.SemaphoreType.DMA((N_DEV,)),
            pltpu.SemaphoreType.DMA((N_DEV,)),
            pltpu.SemaphoreType.DMA((N_DEV,)),
            pltpu.SemaphoreType.DMA((N_DEV,)),
            pltpu.SemaphoreType.DMA((N_DEV,)),
            pltpu.SemaphoreType.DMA((2,)),
            pltpu.SemaphoreType.DMA((2,)),
            pltpu.SemaphoreType.DMA((2,)),
            pltpu.SemaphoreType.DMA((2,)),
        ],
        compiler_params=pltpu.CompilerParams(
            collective_id=0,
            vmem_limit_bytes=58 * 1024 * 1024,
        ),
    )(meta, q, kp, vp, k, v)

    out = ctx.transpose(1, 0, 2).reshape(SQ, HQ * DH) @ Wo
    return out[None]


# device time: 238366 ns/iter; 1.5092x vs baseline; 1.5092x over previous
import functools

import jax
import jax.numpy as jnp
from jax import lax
from jax.experimental import pallas as pl
from jax.experimental.pallas import tpu as pltpu

N_DEV = 8
HQ = 8
DH = 128
SQ = 256
SKV = 4096
QBLK = 64
SCALE = 0.08838834764831843
LOG2E = 1.4426950408889634
SCALE2 = SCALE * LOG2E
NEG = -1e9

RING = (0, 1, 2, 3, 7, 6, 5, 4)
NEXT = (1, 2, 3, 7, 0, 4, 5, 6)
PREV = (4, 0, 1, 2, 5, 6, 7, 3)
POS = (0, 1, 2, 3, 7, 6, 5, 4)


def _body(meta_ref, q_ref, k_ref, v_ref, out_ref,
          qs_ref, acc_ref, st_ref, pv_ref, stl_ref, kbuf, vbuf,
          qsend, qrecv, asend, arecv, ssend, srecv, ksem, vsem):
    right = meta_ref[8]
    left = meta_ref[9]
    my = meta_ref[10]

    qs_ref[0] = q_ref[...]
    acc_ref[0] = jnp.zeros((HQ, SQ, DH), jnp.bfloat16)
    st_ref[0, 0] = jnp.full((SQ, HQ), -1e30, jnp.float32)
    st_ref[0, 1] = jnp.zeros((SQ, HQ), jnp.float32)

    barrier = pltpu.get_barrier_semaphore()
    for nbr in (left, right):
        pl.semaphore_signal(barrier, inc=1, device_id=(nbr,),
                            device_id_type=pl.DeviceIdType.MESH)
    pl.semaphore_wait(barrier, 2)

    row_blk = lax.broadcasted_iota(jnp.int32, (SQ, 1), 0) // QBLK
    col_blk = lax.broadcasted_iota(jnp.int32, (1, SKV), 1) // QBLK
    kb = my * (SKV // QBLK) + col_blk

    def _mk(src, dst, ssem, rsem, dev):
        return pltpu.make_async_remote_copy(
            src_ref=src, dst_ref=dst, send_sem=ssem, recv_sem=rsem,
            device_id=(dev,), device_id_type=pl.DeviceIdType.MESH)

    def _q_hop(s, nxt):
        return _mk(qs_ref.at[s], qs_ref.at[nxt], qsend.at[s],
                   qrecv.at[nxt], right)

    def _acc_hop(s, nxt):
        return (_mk(acc_ref.at[s], acc_ref.at[nxt], asend.at[s],
                    arecv.at[nxt], right),
                _mk(st_ref.at[s], st_ref.at[nxt], ssend.at[s],
                    srecv.at[nxt], right))

    def step(s, carry):
        nxt = (s + 1) % N_DEV

        @pl.when(s > 0)
        def _():
            _q_hop(s - 1, s).wait_send()
            _q_hop(s, s).wait_recv()

        @pl.when(s < N_DEV - 1)
        def _():
            _q_hop(s, nxt).start()

        pltpu.make_async_copy(k_ref.at[0], kbuf.at[0], ksem.at[0]).start()
        pltpu.make_async_copy(v_ref.at[0], vbuf.at[0], vsem.at[0]).start()

        c = meta_ref[s]
        qb = c * (SQ // QBLK) + row_blk
        mask = (qb == kb) | (kb == 0) | ((qb + kb) % 3 == 0)
        bias = jnp.where(mask, 0.0, NEG).astype(jnp.float32)

        for h in range(HQ):
            b = h % 2
            pltpu.make_async_copy(k_ref.at[h], kbuf.at[b], ksem.at[b]).wait()
            pltpu.make_async_copy(v_ref.at[h], vbuf.at[b], vsem.at[b]).wait()
            if h + 1 < HQ:
                nb = (h + 1) % 2
                pltpu.make_async_copy(
                    k_ref.at[h + 1], kbuf.at[nb], ksem.at[nb]).start()
                pltpu.make_async_copy(
                    v_ref.at[h + 1], vbuf.at[nb], vsem.at[nb]).start()
            scores = lax.dot_general(
                qs_ref[s, h], kbuf[b], (((1,), (1,)), ((), ())),
                preferred_element_type=jnp.float32)
            scores = scores * SCALE2 + bias
            m_loc = jnp.max(scores, axis=1, keepdims=True)
            p = jnp.exp2(scores - m_loc)
            stl_ref[0, :, h:h + 1] = m_loc
            stl_ref[1, :, h:h + 1] = jnp.sum(p, axis=1, keepdims=True)
            pv_ref[h] = lax.dot_general(
                p.astype(jnp.bfloat16), vbuf[b], (((1,), (0,)), ((), ())),
                preferred_element_type=jnp.float32).astype(jnp.bfloat16)

        @pl.when(s > 0)
        def _():
            for r in _acc_hop(s - 1, s):
                r.wait_send()
            for r in _acc_hop(s, s):
                r.wait_recv()

        for h in range(HQ):
            m_in = st_ref[s, 0, :, h:h + 1]
            l_in = st_ref[s, 1, :, h:h + 1]
            m_loc = stl_ref[0, :, h:h + 1]
            l_loc = stl_ref[1, :, h:h + 1]
            m_new = jnp.maximum(m_in, m_loc)
            a_in = jnp.exp2(m_in - m_new)
            a_loc = jnp.exp2(m_loc - m_new)
            st_ref[s, 0, :, h:h + 1] = m_new
            st_ref[s, 1, :, h:h + 1] = l_in * a_in + l_loc * a_loc
            acc_ref[s, h] = (
                acc_ref[s, h].astype(jnp.float32) * a_in
                + pv_ref[h].astype(jnp.float32) * a_loc
            ).astype(jnp.bfloat16)

        for r in _acc_hop(s, nxt):
            r.start()
        return carry

    lax.fori_loop(0, N_DEV, step, 0)

    for r in _acc_hop(N_DEV - 1, 0):
        r.wait_send()
    for r in _acc_hop(0, 0):
        r.wait_recv()

    for h in range(HQ):
        out_ref[h] = (acc_ref[0, h].astype(jnp.float32)
                      / st_ref[0, 1, :, h:h + 1])

    @functools.partial(pl.run_scoped, exit_sem=pltpu.SemaphoreType.REGULAR)
    def _(exit_sem):
        for nbr in (left, right):
            pl.semaphore_signal(exit_sem, inc=1, device_id=(nbr,),
                                device_id_type=pl.DeviceIdType.MESH)
        pl.semaphore_wait(exit_sem, 2)


def kernel(x, Wq, K_ext, V_ext, Wo):
    q = (x[0] @ Wq).reshape(SQ, HQ, DH).transpose(1, 0, 2)
    q = q.astype(jnp.bfloat16)
    k = K_ext[0].transpose(1, 0, 2).astype(jnp.bfloat16)
    v = V_ext[0].transpose(1, 0, 2).astype(jnp.bfloat16)

    my = lax.axis_index("i")
    ring = jnp.asarray(RING, jnp.int32)
    pos = jnp.asarray(POS, jnp.int32)[my]
    owners = ring[(pos - jnp.arange(N_DEV, dtype=jnp.int32)) % N_DEV]
    meta = jnp.concatenate([
        owners,
        jnp.asarray(NEXT, jnp.int32)[my][None],
        jnp.asarray(PREV, jnp.int32)[my][None],
        my.astype(jnp.int32)[None],
    ])

    ctx = pl.pallas_call(
        _body,
        out_shape=jax.ShapeDtypeStruct((HQ, SQ, DH), jnp.float32),
        in_specs=[
            pl.BlockSpec(memory_space=pltpu.SMEM),
            pl.BlockSpec(memory_space=pltpu.VMEM),
            pl.BlockSpec(memory_space=pltpu.MemorySpace.HBM),
            pl.BlockSpec(memory_space=pltpu.MemorySpace.HBM),
        ],
        out_specs=pl.BlockSpec(memory_space=pltpu.VMEM),
        scratch_shapes=[
            pltpu.VMEM((N_DEV, HQ, SQ, DH), jnp.bfloat16),
            pltpu.VMEM((N_DEV, HQ, SQ, DH), jnp.bfloat16),
            pltpu.VMEM((N_DEV, 2, SQ, HQ), jnp.float32),
            pltpu.VMEM((HQ, SQ, DH), jnp.bfloat16),
            pltpu.VMEM((2, SQ, HQ), jnp.float32),
            pltpu.VMEM((2, SKV, DH), jnp.bfloat16),
            pltpu.VMEM((2, SKV, DH), jnp.bfloat16),
            pltpu.SemaphoreType.DMA((N_DEV,)),
            pltpu.SemaphoreType.DMA((N_DEV,)),
            pltpu.SemaphoreType.DMA((N_DEV,)),
            pltpu.SemaphoreType.DMA((N_DEV,)),
            pltpu.SemaphoreType.DMA((N_DEV,)),
            pltpu.SemaphoreType.DMA((N_DEV,)),
            pltpu.SemaphoreType.DMA((2,)),
            pltpu.SemaphoreType.DMA((2,)),
        ],
        compiler_params=pltpu.CompilerParams(
            collective_id=0,
            vmem_limit_bytes=58 * 1024 * 1024,
        ),
    )(meta, q, k, v)

    out = ctx.transpose(1, 0, 2).reshape(SQ, HQ * DH) @ Wo
    return out[None]


# device time: 203808 ns/iter; 1.7651x vs baseline; 1.1696x over previous
import functools

import jax
import jax.numpy as jnp
from jax import lax
from jax.experimental import pallas as pl
from jax.experimental.pallas import tpu as pltpu

N_DEV = 8
HQ = 8
DH = 128
SQ = 256
SKV = 4096
QBLK = 64
SCALE = 0.08838834764831843
LOG2E = 1.4426950408889634
SCALE2 = SCALE * LOG2E
NEG = -1e9

RING = (0, 1, 2, 3, 7, 6, 5, 4)
NEXT = (1, 2, 3, 7, 0, 4, 5, 6)
PREV = (4, 0, 1, 2, 5, 6, 7, 3)
POS = (0, 1, 2, 3, 7, 6, 5, 4)


def _body(meta_ref, q_ref, k_ref, v_ref, out_ref,
          qs_ref, acc_ref, st_ref, pv_ref, stl_ref, kbuf, vbuf,
          qsend, qrecv, asend, arecv, ssend, srecv, ksem, vsem):
    right = meta_ref[8]
    left = meta_ref[9]
    my = meta_ref[10]

    qs_ref[0] = q_ref[...]
    acc_ref[0] = jnp.zeros((HQ, SQ, DH), jnp.bfloat16)
    st_ref[0] = jnp.zeros((SQ, HQ), jnp.float32)

    barrier = pltpu.get_barrier_semaphore()
    for nbr in (left, right):
        pl.semaphore_signal(barrier, inc=1, device_id=(nbr,),
                            device_id_type=pl.DeviceIdType.MESH)
    pl.semaphore_wait(barrier, 2)

    row_blk = lax.broadcasted_iota(jnp.int32, (SQ, 1), 0) // QBLK
    col_blk = lax.broadcasted_iota(jnp.int32, (1, SKV), 1) // QBLK
    kb = my * (SKV // QBLK) + col_blk

    def _mk(src, dst, ssem, rsem, dev):
        return pltpu.make_async_remote_copy(
            src_ref=src, dst_ref=dst, send_sem=ssem, recv_sem=rsem,
            device_id=(dev,), device_id_type=pl.DeviceIdType.MESH)

    def _q_hop(s, nxt):
        return _mk(qs_ref.at[s], qs_ref.at[nxt], qsend.at[s],
                   qrecv.at[nxt], right)

    def _acc_hop(s, nxt):
        return (_mk(acc_ref.at[s], acc_ref.at[nxt], asend.at[s],
                    arecv.at[nxt], right),
                _mk(st_ref.at[s], st_ref.at[nxt], ssend.at[s],
                    srecv.at[nxt], right))

    def step(s, carry):
        nxt = (s + 1) % N_DEV

        @pl.when(s > 0)
        def _():
            _q_hop(s - 1, s).wait_send()
            _q_hop(s, s).wait_recv()

        @pl.when(s < N_DEV - 1)
        def _():
            _q_hop(s, nxt).start()

        pltpu.make_async_copy(k_ref.at[0], kbuf.at[0], ksem.at[0]).start()
        pltpu.make_async_copy(v_ref.at[0], vbuf.at[0], vsem.at[0]).start()

        c = meta_ref[s]
        qb = c * (SQ // QBLK) + row_blk
        mask = (qb == kb) | (kb == 0) | ((qb + kb) % 3 == 0)
        bias = jnp.where(mask, 0.0, NEG).astype(jnp.float32)

        for h in range(HQ):
            b = h % 2
            pltpu.make_async_copy(k_ref.at[h], kbuf.at[b], ksem.at[b]).wait()
            pltpu.make_async_copy(v_ref.at[h], vbuf.at[b], vsem.at[b]).wait()
            if h + 1 < HQ:
                nb = (h + 1) % 2
                pltpu.make_async_copy(
                    k_ref.at[h + 1], kbuf.at[nb], ksem.at[nb]).start()
                pltpu.make_async_copy(
                    v_ref.at[h + 1], vbuf.at[nb], vsem.at[nb]).start()
            scores = lax.dot_general(
                qs_ref[s, h], kbuf[b], (((1,), (1,)), ((), ())),
                preferred_element_type=jnp.float32)
            p = jnp.exp2(scores * SCALE2 + bias)
            stl_ref[:, h:h + 1] = jnp.sum(p, axis=1, keepdims=True)
            pv_ref[h] = lax.dot_general(
                p.astype(jnp.bfloat16), vbuf[b], (((1,), (0,)), ((), ())),
                preferred_element_type=jnp.float32).astype(jnp.bfloat16)

        @pl.when(s > 0)
        def _():
            for r in _acc_hop(s - 1, s):
                r.wait_send()
            for r in _acc_hop(s, s):
                r.wait_recv()

        for h in range(HQ):
            st_ref[s, :, h:h + 1] = (st_ref[s, :, h:h + 1]
                                     + stl_ref[:, h:h + 1])
            acc_ref[s, h] = (
                acc_ref[s, h].astype(jnp.float32)
                + pv_ref[h].astype(jnp.float32)
            ).astype(jnp.bfloat16)

        for r in _acc_hop(s, nxt):
            r.start()
        return carry

    lax.fori_loop(0, N_DEV, step, 0)

    for r in _acc_hop(N_DEV - 1, 0):
        r.wait_send()
    for r in _acc_hop(0, 0):
        r.wait_recv()

    for h in range(HQ):
        out_ref[h] = (acc_ref[0, h].astype(jnp.float32)
                      / st_ref[0, :, h:h + 1])

    @functools.partial(pl.run_scoped, exit_sem=pltpu.SemaphoreType.REGULAR)
    def _(exit_sem):
        for nbr in (left, right):
            pl.semaphore_signal(exit_sem, inc=1, device_id=(nbr,),
                                device_id_type=pl.DeviceIdType.MESH)
        pl.semaphore_wait(exit_sem, 2)


def kernel(x, Wq, K_ext, V_ext, Wo):
    q = (x[0] @ Wq).reshape(SQ, HQ, DH).transpose(1, 0, 2)
    q = q.astype(jnp.bfloat16)
    k = K_ext[0].transpose(1, 0, 2).astype(jnp.bfloat16)
    v = V_ext[0].transpose(1, 0, 2).astype(jnp.bfloat16)

    my = lax.axis_index("i")
    ring = jnp.asarray(RING, jnp.int32)
    pos = jnp.asarray(POS, jnp.int32)[my]
    owners = ring[(pos - jnp.arange(N_DEV, dtype=jnp.int32)) % N_DEV]
    meta = jnp.concatenate([
        owners,
        jnp.asarray(NEXT, jnp.int32)[my][None],
        jnp.asarray(PREV, jnp.int32)[my][None],
        my.astype(jnp.int32)[None],
    ])

    ctx = pl.pallas_call(
        _body,
        out_shape=jax.ShapeDtypeStruct((HQ, SQ, DH), jnp.float32),
        in_specs=[
            pl.BlockSpec(memory_space=pltpu.SMEM),
            pl.BlockSpec(memory_space=pltpu.VMEM),
            pl.BlockSpec(memory_space=pltpu.MemorySpace.HBM),
            pl.BlockSpec(memory_space=pltpu.MemorySpace.HBM),
        ],
        out_specs=pl.BlockSpec(memory_space=pltpu.VMEM),
        scratch_shapes=[
            pltpu.VMEM((N_DEV, HQ, SQ, DH), jnp.bfloat16),
            pltpu.VMEM((N_DEV, HQ, SQ, DH), jnp.bfloat16),
            pltpu.VMEM((N_DEV, SQ, HQ), jnp.float32),
            pltpu.VMEM((HQ, SQ, DH), jnp.bfloat16),
            pltpu.VMEM((SQ, HQ), jnp.float32),
            pltpu.VMEM((2, SKV, DH), jnp.bfloat16),
            pltpu.VMEM((2, SKV, DH), jnp.bfloat16),
            pltpu.SemaphoreType.DMA((N_DEV,)),
            pltpu.SemaphoreType.DMA((N_DEV,)),
            pltpu.SemaphoreType.DMA((N_DEV,)),
            pltpu.SemaphoreType.DMA((N_DEV,)),
            pltpu.SemaphoreType.DMA((N_DEV,)),
            pltpu.SemaphoreType.DMA((N_DEV,)),
            pltpu.SemaphoreType.DMA((2,)),
            pltpu.SemaphoreType.DMA((2,)),
        ],
        compiler_params=pltpu.CompilerParams(
            collective_id=0,
            vmem_limit_bytes=58 * 1024 * 1024,
        ),
    )(meta, q, k, v)

    out = ctx.transpose(1, 0, 2).reshape(SQ, HQ * DH) @ Wo
    return out[None]
